# baseline (speedup 1.0000x reference)
import functools

import jax
import jax.numpy as jnp
from jax import lax
from jax.experimental import pallas as pl
from jax.experimental.pallas import tpu as pltpu
from jax.experimental.pallas import tpu_sc as plsc

N_PAD = 102400
NC = 2
NS = 16
NW = NC * NS
LANES = 16
CHUNK = N_PAD // NW
NVEC = CHUNK // LANES
TBL = 16


def _sc_body(ae_hbm, idx_hbm, scale_hbm, shift_hbm, out_hbm,
             ae_v, idx_v, out_v, scale_v, shift_v):
    wid = lax.axis_index("s") * NC + lax.axis_index("c")
    base = wid * CHUNK
    pltpu.sync_copy(scale_hbm, scale_v)
    pltpu.sync_copy(shift_hbm, shift_v)
    pltpu.sync_copy(idx_hbm.at[pl.ds(base, CHUNK)], idx_v)
    pltpu.sync_copy(ae_hbm.at[pl.ds(base, CHUNK)], ae_v)

    def step(i, carry):
        s = pl.ds(i * LANES, LANES)
        iv = idx_v[s]
        av = ae_v[s]
        sc = plsc.load_gather(scale_v, [iv])
        sh = plsc.load_gather(shift_v, [iv])
        out_v[s] = av * sc + sh
        return carry

    lax.fori_loop(0, NVEC, step, 0, unroll=4)
    pltpu.sync_copy(out_v, out_hbm.at[pl.ds(base, CHUNK)])


_sc_call = pl.kernel(
    _sc_body,
    out_type=jax.ShapeDtypeStruct((N_PAD,), jnp.float32),
    mesh=plsc.VectorSubcoreMesh(
        core_axis_name="c", subcore_axis_name="s",
        num_cores=NC, num_subcores=NS),
    compiler_params=pltpu.CompilerParams(needs_layout_passes=False),
    scratch_types=[
        pltpu.VMEM((CHUNK,), jnp.float32),
        pltpu.VMEM((CHUNK,), jnp.int32),
        pltpu.VMEM((CHUNK,), jnp.float32),
        pltpu.VMEM((TBL,), jnp.float32),
        pltpu.VMEM((TBL,), jnp.float32),
    ],
)


def kernel(atomic_energy, atom_number, scale, shift):
    ae = atomic_energy.reshape(-1).astype(jnp.float32)
    n = ae.shape[0]
    idx = atom_number.reshape(-1).astype(jnp.int32)
    ae_p = jnp.pad(ae, (0, N_PAD - n))
    idx_p = jnp.pad(idx, (0, N_PAD - n))
    scale_p = jnp.pad(scale.astype(jnp.float32), (0, TBL - scale.shape[0]))
    shift_p = jnp.pad(shift.astype(jnp.float32), (0, TBL - shift.shape[0]))
    out = _sc_call(ae_p, idx_p, scale_p, shift_p)
    return out[:n]


# speedup vs baseline: 1.1033x; 1.1033x over previous
import jax
import jax.numpy as jnp
from jax import lax
from jax.experimental import pallas as pl
from jax.experimental.pallas import tpu as pltpu
from jax.experimental.pallas import tpu_sc as plsc

N = 100000
NC = 2
NS = 16
NW = NC * NS
LANES = 16
CHUNK = 3120
NBIG = 10
CHUNK_BIG = CHUNK + LANES
NVEC = CHUNK // LANES
TBL = 16


def _sc_body(ae_hbm, idx_hbm, scale_hbm, shift_hbm, out_hbm,
             ae_v, idx_v, out_v, scale_v, shift_v):
    wid = lax.axis_index("s") * NC + lax.axis_index("c")
    base = CHUNK * wid + LANES * jnp.minimum(wid, NBIG)
    big = wid < NBIG
    pltpu.sync_copy(scale_hbm, scale_v)
    pltpu.sync_copy(shift_hbm, shift_v)
    pltpu.sync_copy(idx_hbm.at[pl.ds(base, CHUNK)], idx_v.at[pl.ds(0, CHUNK)])
    pltpu.sync_copy(ae_hbm.at[pl.ds(base, CHUNK)], ae_v.at[pl.ds(0, CHUNK)])

    @pl.when(big)
    def _():
        pltpu.sync_copy(idx_hbm.at[pl.ds(base + CHUNK, LANES)],
                        idx_v.at[pl.ds(CHUNK, LANES)])
        pltpu.sync_copy(ae_hbm.at[pl.ds(base + CHUNK, LANES)],
                        ae_v.at[pl.ds(CHUNK, LANES)])

    def step(i, carry):
        s = pl.ds(i * LANES, LANES)
        iv = idx_v[s]
        av = ae_v[s]
        sc = plsc.load_gather(scale_v, [iv])
        sh = plsc.load_gather(shift_v, [iv])
        out_v[s] = av * sc + sh
        return carry

    lax.fori_loop(0, NVEC, step, 0, unroll=8)

    @pl.when(big)
    def _():
        step(NVEC, 0)

    pltpu.sync_copy(out_v.at[pl.ds(0, CHUNK)], out_hbm.at[pl.ds(base, CHUNK)])

    @pl.when(big)
    def _():
        pltpu.sync_copy(out_v.at[pl.ds(CHUNK, LANES)],
                        out_hbm.at[pl.ds(base + CHUNK, LANES)])


_sc_call = pl.kernel(
    _sc_body,
    out_type=jax.ShapeDtypeStruct((N,), jnp.float32),
    mesh=plsc.VectorSubcoreMesh(
        core_axis_name="c", subcore_axis_name="s",
        num_cores=NC, num_subcores=NS),
    compiler_params=pltpu.CompilerParams(needs_layout_passes=False),
    scratch_types=[
        pltpu.VMEM((CHUNK_BIG,), jnp.float32),
        pltpu.VMEM((CHUNK_BIG,), jnp.int32),
        pltpu.VMEM((CHUNK_BIG,), jnp.float32),
        pltpu.VMEM((TBL,), jnp.float32),
        pltpu.VMEM((TBL,), jnp.float32),
    ],
)


def kernel(atomic_energy, atom_number, scale, shift):
    ae = atomic_energy.reshape(-1).astype(jnp.float32)
    idx = atom_number.reshape(-1).astype(jnp.int32)
    scale_p = jnp.pad(scale.astype(jnp.float32), (0, TBL - scale.shape[0]))
    shift_p = jnp.pad(shift.astype(jnp.float32), (0, TBL - shift.shape[0]))
    return _sc_call(ae, idx, scale_p, shift_p)


# speedup vs baseline: 1.3844x; 1.2548x over previous
import jax
import jax.numpy as jnp
from jax import lax
from jax.experimental import pallas as pl
from jax.experimental.pallas import tpu as pltpu
from jax.experimental.pallas import tpu_sc as plsc

N = 100000
NC = 2
NS = 16
NW = NC * NS
LANES = 16
CHUNK = 3136
LAST = N - CHUNK
NSP = 10


def _sc_body(ae_hbm, idx_hbm, scale_hbm, shift_hbm, out_hbm,
             ae_v, idx_v, out_v, scale_v, shift_v, sem):
    wid = lax.axis_index("s") * NC + lax.axis_index("c")
    base = jnp.minimum(CHUNK * wid, LAST)
    c1 = pltpu.make_async_copy(idx_hbm.at[pl.ds(base, CHUNK)], idx_v, sem)
    c2 = pltpu.make_async_copy(ae_hbm.at[pl.ds(base, CHUNK)], ae_v, sem)
    c3 = pltpu.make_async_copy(scale_hbm, scale_v.at[pl.ds(0, NSP)], sem)
    c4 = pltpu.make_async_copy(shift_hbm, shift_v.at[pl.ds(0, NSP)], sem)
    c1.start(); c2.start(); c3.start(); c4.start()
    c1.wait(); c2.wait(); c3.wait(); c4.wait()

    @plsc.parallel_loop(0, CHUNK, step=LANES, unroll=4)
    def _(off):
        s = pl.ds(off, LANES)
        iv = idx_v[s]
        av = ae_v[s]
        sc = plsc.load_gather(scale_v, [iv])
        sh = plsc.load_gather(shift_v, [iv])
        out_v[s] = av * sc + sh

    pltpu.sync_copy(out_v, out_hbm.at[pl.ds(base, CHUNK)])


_sc_call = pl.kernel(
    _sc_body,
    out_type=jax.ShapeDtypeStruct((N,), jnp.float32),
    mesh=plsc.VectorSubcoreMesh(
        core_axis_name="c", subcore_axis_name="s",
        num_cores=NC, num_subcores=NS),
    compiler_params=pltpu.CompilerParams(needs_layout_passes=False),
    scratch_types=[
        pltpu.VMEM((CHUNK,), jnp.float32),
        pltpu.VMEM((CHUNK,), jnp.int32),
        pltpu.VMEM((CHUNK,), jnp.float32),
        pltpu.VMEM((LANES,), jnp.float32),
        pltpu.VMEM((LANES,), jnp.float32),
        pltpu.SemaphoreType.DMA,
    ],
)


def kernel(atomic_energy, atom_number, scale, shift):
    ae = atomic_energy.reshape(-1).astype(jnp.float32)
    idx = atom_number.reshape(-1).astype(jnp.int32)
    return _sc_call(ae, idx, scale.astype(jnp.float32),
                    shift.astype(jnp.float32))
